# baseline (speedup 1.0000x reference)
import functools

import jax
import jax.numpy as jnp
from jax import lax
from jax.experimental import pallas as pl
from jax.experimental.pallas import tpu as pltpu
from jax.experimental.pallas import tpu_sc as plsc

N = 50000
B = 64
D = 256
H = 512

NC = 2
NS = 16
NW = NC * NS

C_BLK = 80
N_SC_BLK = N // C_BLK
V_MAX = (N_SC_BLK + NW - 1) // NW


def _sc_segsum_body(nfeats_hbm, bid_hbm, pooled_hbm, buf0, buf1, idx0, idx1,
                    acc, sem0, sem1):
    cid = lax.axis_index("c")
    sid = lax.axis_index("s")
    wid = sid * NC + cid

    def zero_chunk(t, carry):
        acc[pl.ds(t * 16, 16)] = jnp.zeros((16,), jnp.float32)
        return carry

    lax.fori_loop(0, B * D // 16, zero_chunk, 0)

    nv = (N_SC_BLK - wid + NW - 1) // NW

    def start(v, buf, idxv, sem):
        base = (wid + v * NW) * C_BLK
        pltpu.async_copy(nfeats_hbm.at[pl.ds(base * D, C_BLK * D)], buf, sem)
        pltpu.async_copy(bid_hbm.at[pl.ds(base, C_BLK)], idxv, sem)

    def wait(buf, idxv, sem):
        pltpu.make_async_copy(nfeats_hbm.at[pl.ds(0, C_BLK * D)], buf,
                              sem).wait()
        pltpu.make_async_copy(bid_hbm.at[pl.ds(0, C_BLK)], idxv, sem).wait()

    NJ = D // 16

    def flush(cur, regs):
        base = cur * D
        for j in range(NJ):
            plsc.addupdate(acc.at[pl.ds(base + j * 16, 16)], regs[j])

    def process(buf, idxv):
        def grp(g, c):
            cur = c[0]
            regs = list(c[1:])
            bids = idxv[pl.ds(g * 16, 16)]
            b0 = bids[0]
            b15 = bids[15]
            uniform = b0 == b15
            same = jnp.logical_and(uniform, b0 == cur)

            @pl.when(jnp.logical_and(uniform, b0 != cur))
            def _boundary_flush():
                flush(cur, regs)

            @pl.when(jnp.logical_not(uniform))
            def _mixed():
                flush(cur, regs)
                for lane in range(16):
                    rbase = (g * 16 + lane) * D
                    abase = bids[lane] * D
                    for j in range(NJ):
                        plsc.addupdate(acc.at[pl.ds(abase + j * 16, 16)],
                                       buf[pl.ds(rbase + j * 16, 16)])

            gbase = g * 16 * D
            s = [buf[pl.ds(gbase + j * 16, 16)] for j in range(NJ)]
            for r in range(1, 16):
                for j in range(NJ):
                    s[j] = s[j] + buf[pl.ds(gbase + r * D + j * 16, 16)]

            zero = jnp.zeros((16,), jnp.float32)
            new_regs = tuple(
                jnp.where(same, regs[j] + s[j], jnp.where(uniform, s[j], zero))
                for j in range(NJ)
            )
            cur_new = jnp.where(uniform, b0, b15)
            return (cur_new,) + new_regs

        zero16 = jnp.zeros((16,), jnp.float32)
        carry0 = (jnp.int32(0),) + tuple(zero16 for _ in range(NJ))
        carry = lax.fori_loop(0, C_BLK // 16, grp, carry0)
        flush(carry[0], carry[1:])

    start(0, buf0, idx0, sem0)

    def pair(k, carry):
        i = 2 * k

        @pl.when(i + 1 < nv)
        def _start_odd():
            start(i + 1, buf1, idx1, sem1)

        wait(buf0, idx0, sem0)
        process(buf0, idx0)

        @pl.when(i + 2 < nv)
        def _start_even():
            start(i + 2, buf0, idx0, sem0)

        @pl.when(i + 1 < nv)
        def _process_odd():
            wait(buf1, idx1, sem1)
            process(buf1, idx1)

        return carry

    lax.fori_loop(0, V_MAX // 2, pair, 0)

    pltpu.sync_copy(acc, pooled_hbm.at[wid])


_sc_segsum = functools.partial(
    pl.kernel,
    out_type=jax.ShapeDtypeStruct((NW, B * D), jnp.float32),
    mesh=plsc.VectorSubcoreMesh(
        core_axis_name="c", subcore_axis_name="s",
        num_cores=NC, num_subcores=NS,
    ),
    scratch_types=[
        pltpu.VMEM((C_BLK * D,), jnp.float32),
        pltpu.VMEM((C_BLK * D,), jnp.float32),
        pltpu.VMEM((C_BLK,), jnp.int32),
        pltpu.VMEM((C_BLK,), jnp.int32),
        pltpu.VMEM((B * D,), jnp.float32),
        pltpu.SemaphoreType.DMA,
        pltpu.SemaphoreType.DMA,
    ],
)(_sc_segsum_body)


R_BLK = 2000
N_TC_BLK = N // R_BLK


def _tc_add_body(bid_ref, nfeats_ref, emb_ref, out_ref, counts_ref):
    i = pl.program_id(0)
    out_ref[...] = nfeats_ref[...] + emb_ref[...]
    ids = bid_ref[0]
    g = lax.broadcasted_iota(jnp.int32, (B, R_BLK), 0)
    c = jnp.sum((ids == g).astype(jnp.float32), axis=1)

    @pl.when(i == 0)
    def _init():
        counts_ref[...] = c[None, :]

    @pl.when(i > 0)
    def _accum():
        counts_ref[...] = counts_ref[...] + c[None, :]


def _tc_mlp_body(pooled_ref, counts_ref, emb_ref, w1_ref, b1_ref, w2_ref,
                 b2_ref, out_ref):
    e = emb_ref[...]
    pooled = jnp.sum(pooled_ref[...], axis=0)
    v = pooled + counts_ref[0][:, None] * e + e
    h = jnp.dot(v, w1_ref[...], preferred_element_type=jnp.float32)
    h = jnp.maximum(h + b1_ref[...], 0.0)
    o = jnp.dot(h, w2_ref[...], preferred_element_type=jnp.float32)
    out_ref[...] = o + b2_ref[...] + e


def kernel(nfeats, batch_id, init_emb, W1, b1, W2, b2):
    bid = batch_id.astype(jnp.int32)

    pooled_raw = _sc_segsum(nfeats.reshape(N * D), bid).reshape(NW, B, D)

    bid3 = bid.reshape(N_TC_BLK, 1, R_BLK)
    nfeats_out, counts = pl.pallas_call(
        _tc_add_body,
        grid=(N_TC_BLK,),
        in_specs=[
            pl.BlockSpec((1, 1, R_BLK), lambda i: (i, 0, 0)),
            pl.BlockSpec((R_BLK, D), lambda i: (i, 0)),
            pl.BlockSpec((1, D), lambda i: (0, 0)),
        ],
        out_specs=[
            pl.BlockSpec((R_BLK, D), lambda i: (i, 0)),
            pl.BlockSpec((1, B), lambda i: (0, 0)),
        ],
        out_shape=[
            jax.ShapeDtypeStruct((N, D), jnp.float32),
            jax.ShapeDtypeStruct((1, B), jnp.float32),
        ],
    )(bid3, nfeats, init_emb)

    vnfeat_out = pl.pallas_call(
        _tc_mlp_body,
        out_shape=jax.ShapeDtypeStruct((B, D), jnp.float32),
    )(pooled_raw, counts, init_emb, W1, b1.reshape(1, H), W2,
      b2.reshape(1, D))

    return nfeats_out, vnfeat_out


# speedup vs baseline: 1.7529x; 1.7529x over previous
import functools

import jax
import jax.numpy as jnp
from jax import lax
from jax.experimental import pallas as pl
from jax.experimental.pallas import tpu as pltpu
from jax.experimental.pallas import tpu_sc as plsc

N = 50000
B = 64
D = 256
H = 512

N_SC = 22000

NC = 2
NS = 16
NW = NC * NS

C_BLK = 80
N_SC_BLK = N_SC // C_BLK
V_MAX = (N_SC_BLK + NW - 1) // NW


def _sc_segsum_body(nfeats_hbm, bid_hbm, pooled_hbm, buf0, buf1, idx0, idx1,
                    acc, sem0, sem1):
    cid = lax.axis_index("c")
    sid = lax.axis_index("s")
    wid = sid * NC + cid

    def zero_row(r, carry):
        for j in range(D // 16):
            acc[r, pl.ds(j * 16, 16)] = jnp.zeros((16,), jnp.float32)
        return carry

    lax.fori_loop(0, B, zero_row, 0)

    nv = (N_SC_BLK - wid + NW - 1) // NW

    def start(v, buf, idxv, sem):
        base = (wid + v * NW) * C_BLK
        pltpu.async_copy(nfeats_hbm.at[pl.ds(base, C_BLK)], buf, sem)
        pltpu.async_copy(bid_hbm.at[pl.ds(base, C_BLK)], idxv, sem)

    def wait(buf, idxv, sem):
        pltpu.make_async_copy(nfeats_hbm.at[pl.ds(0, C_BLK)], buf, sem).wait()
        pltpu.make_async_copy(bid_hbm.at[pl.ds(0, C_BLK)], idxv, sem).wait()

    def process(buf, idxv):
        def grp(g, carry2):
            bids = idxv[pl.ds(g * 16, 16)]
            for lane in range(16):
                b = bids[lane]
                r = g * 16 + lane
                for j in range(D // 16):
                    plsc.addupdate(acc.at[b, pl.ds(j * 16, 16)],
                                   buf[r, pl.ds(j * 16, 16)])
            return carry2

        lax.fori_loop(0, C_BLK // 16, grp, 0)

    start(0, buf0, idx0, sem0)

    def pair(k, carry):
        i = 2 * k

        @pl.when(i + 1 < nv)
        def _start_odd():
            start(i + 1, buf1, idx1, sem1)

        @pl.when(i < nv)
        def _process_even():
            wait(buf0, idx0, sem0)
            process(buf0, idx0)

        @pl.when(i + 2 < nv)
        def _start_even():
            start(i + 2, buf0, idx0, sem0)

        @pl.when(i + 1 < nv)
        def _process_odd():
            wait(buf1, idx1, sem1)
            process(buf1, idx1)

        return carry

    lax.fori_loop(0, (V_MAX + 1) // 2, pair, 0)

    pltpu.sync_copy(acc, pooled_hbm.at[wid])


_sc_segsum = functools.partial(
    pl.kernel,
    out_type=jax.ShapeDtypeStruct((NW, B, D), jnp.float32),
    mesh=plsc.VectorSubcoreMesh(
        core_axis_name="c", subcore_axis_name="s",
        num_cores=NC, num_subcores=NS,
    ),
    scratch_types=[
        pltpu.VMEM((C_BLK, D), jnp.float32),
        pltpu.VMEM((C_BLK, D), jnp.float32),
        pltpu.VMEM((C_BLK,), jnp.int32),
        pltpu.VMEM((C_BLK,), jnp.int32),
        pltpu.VMEM((B, D), jnp.float32),
        pltpu.SemaphoreType.DMA,
        pltpu.SemaphoreType.DMA,
    ],
)(_sc_segsum_body)


R_BLK = 2000
N_TC_BLK = N // R_BLK
TC_POOL_START = N_SC // R_BLK


def _tc_add_body(bid_ref, nfeats_ref, emb_ref, out_ref, counts_ref, ptc_ref):
    i = pl.program_id(0)
    out_ref[...] = nfeats_ref[...] + emb_ref[...]
    ids = bid_ref[0]
    g = lax.broadcasted_iota(jnp.int32, (B, R_BLK), 0)
    onehot = (ids == g).astype(jnp.float32)
    c = jnp.sum(onehot, axis=1)

    @pl.when(i == 0)
    def _init_counts():
        counts_ref[...] = c[None, :]

    @pl.when(i > 0)
    def _accum_counts():
        counts_ref[...] = counts_ref[...] + c[None, :]

    @pl.when(i == TC_POOL_START)
    def _init_pool():
        ptc_ref[...] = jnp.dot(onehot, nfeats_ref[...],
                               preferred_element_type=jnp.float32)

    @pl.when(i > TC_POOL_START)
    def _accum_pool():
        ptc_ref[...] = ptc_ref[...] + jnp.dot(
            onehot, nfeats_ref[...], preferred_element_type=jnp.float32)


def _tc_mlp_body(pooled_ref, ptc_ref, counts_ref, emb_ref, w1_ref, b1_ref,
                 w2_ref, b2_ref, out_ref):
    e = emb_ref[...]
    pooled = jnp.sum(pooled_ref[...], axis=0) + ptc_ref[...]
    v = pooled + counts_ref[0][:, None] * e + e
    h = jnp.dot(v, w1_ref[...], preferred_element_type=jnp.float32)
    h = jnp.maximum(h + b1_ref[...], 0.0)
    o = jnp.dot(h, w2_ref[...], preferred_element_type=jnp.float32)
    out_ref[...] = o + b2_ref[...] + e


def kernel(nfeats, batch_id, init_emb, W1, b1, W2, b2):
    bid = batch_id.astype(jnp.int32)

    pooled_sc = _sc_segsum(nfeats, bid)

    bid3 = bid.reshape(N_TC_BLK, 1, R_BLK)
    nfeats_out, counts, pooled_tc = pl.pallas_call(
        _tc_add_body,
        grid=(N_TC_BLK,),
        in_specs=[
            pl.BlockSpec((1, 1, R_BLK), lambda i: (i, 0, 0)),
            pl.BlockSpec((R_BLK, D), lambda i: (i, 0)),
            pl.BlockSpec((1, D), lambda i: (0, 0)),
        ],
        out_specs=[
            pl.BlockSpec((R_BLK, D), lambda i: (i, 0)),
            pl.BlockSpec((1, B), lambda i: (0, 0)),
            pl.BlockSpec((B, D), lambda i: (0, 0)),
        ],
        out_shape=[
            jax.ShapeDtypeStruct((N, D), jnp.float32),
            jax.ShapeDtypeStruct((1, B), jnp.float32),
            jax.ShapeDtypeStruct((B, D), jnp.float32),
        ],
    )(bid3, nfeats, init_emb)

    vnfeat_out = pl.pallas_call(
        _tc_mlp_body,
        out_shape=jax.ShapeDtypeStruct((B, D), jnp.float32),
    )(pooled_sc, pooled_tc, counts, init_emb, W1, b1.reshape(1, H), W2,
      b2.reshape(1, D))

    return nfeats_out, vnfeat_out


# speedup vs baseline: 1.8668x; 1.0650x over previous
import functools

import jax
import jax.numpy as jnp
from jax import lax
from jax.experimental import pallas as pl
from jax.experimental.pallas import tpu as pltpu
from jax.experimental.pallas import tpu_sc as plsc

N = 50000
B = 64
D = 256
H = 512

N_SC = 20000

NC = 2
NS = 16
NW = NC * NS

C_BLK = 80
N_SC_BLK = N_SC // C_BLK
V_MAX = (N_SC_BLK + NW - 1) // NW


def _sc_segsum_body(nfeats_hbm, bid_hbm, pooled_hbm, buf0, buf1, idx0, idx1,
                    acc, sem0, sem1):
    cid = lax.axis_index("c")
    sid = lax.axis_index("s")
    wid = sid * NC + cid

    def zero_row(r, carry):
        for j in range(D // 16):
            acc[r, pl.ds(j * 16, 16)] = jnp.zeros((16,), jnp.float32)
        return carry

    lax.fori_loop(0, B, zero_row, 0)

    nv = (N_SC_BLK - wid + NW - 1) // NW

    def start(v, buf, idxv, sem):
        base = (wid + v * NW) * C_BLK
        pltpu.async_copy(nfeats_hbm.at[pl.ds(base, C_BLK)], buf, sem)
        pltpu.async_copy(bid_hbm.at[pl.ds(base, C_BLK)], idxv, sem)

    def wait(buf, idxv, sem):
        pltpu.make_async_copy(nfeats_hbm.at[pl.ds(0, C_BLK)], buf, sem).wait()
        pltpu.make_async_copy(bid_hbm.at[pl.ds(0, C_BLK)], idxv, sem).wait()

    def process(buf, idxv):
        def grp(g, carry2):
            bids = idxv[pl.ds(g * 16, 16)]
            for lane in range(16):
                b = bids[lane]
                r = g * 16 + lane
                for j in range(D // 16):
                    plsc.addupdate(acc.at[b, pl.ds(j * 16, 16)],
                                   buf[r, pl.ds(j * 16, 16)])
            return carry2

        lax.fori_loop(0, C_BLK // 16, grp, 0)

    start(0, buf0, idx0, sem0)

    def pair(k, carry):
        i = 2 * k

        @pl.when(i + 1 < nv)
        def _start_odd():
            start(i + 1, buf1, idx1, sem1)

        @pl.when(i < nv)
        def _process_even():
            wait(buf0, idx0, sem0)
            process(buf0, idx0)

        @pl.when(i + 2 < nv)
        def _start_even():
            start(i + 2, buf0, idx0, sem0)

        @pl.when(i + 1 < nv)
        def _process_odd():
            wait(buf1, idx1, sem1)
            process(buf1, idx1)

        return carry

    lax.fori_loop(0, (V_MAX + 1) // 2, pair, 0)

    pltpu.sync_copy(acc, pooled_hbm.at[wid])


_sc_segsum = functools.partial(
    pl.kernel,
    out_type=jax.ShapeDtypeStruct((NW, B, D), jnp.float32),
    mesh=plsc.VectorSubcoreMesh(
        core_axis_name="c", subcore_axis_name="s",
        num_cores=NC, num_subcores=NS,
    ),
    scratch_types=[
        pltpu.VMEM((C_BLK, D), jnp.float32),
        pltpu.VMEM((C_BLK, D), jnp.float32),
        pltpu.VMEM((C_BLK,), jnp.int32),
        pltpu.VMEM((C_BLK,), jnp.int32),
        pltpu.VMEM((B, D), jnp.float32),
        pltpu.SemaphoreType.DMA,
        pltpu.SemaphoreType.DMA,
    ],
)(_sc_segsum_body)


R_BLK = 2000
N_TC_BLK = N // R_BLK
TC_POOL_START = N_SC // R_BLK


def _tc_add_body(bid_ref, nfeats_ref, emb_ref, out_ref, counts_ref, ptc_ref):
    i = pl.program_id(0)
    out_ref[...] = nfeats_ref[...] + emb_ref[...]
    ids = bid_ref[0]
    g = lax.broadcasted_iota(jnp.int32, (B, R_BLK), 0)
    onehot = (ids == g).astype(jnp.float32)
    c = jnp.sum(onehot, axis=1)

    @pl.when(i == 0)
    def _init_counts():
        counts_ref[...] = c[None, :]

    @pl.when(i > 0)
    def _accum_counts():
        counts_ref[...] = counts_ref[...] + c[None, :]

    @pl.when(i == TC_POOL_START)
    def _init_pool():
        ptc_ref[...] = jnp.dot(onehot, nfeats_ref[...],
                               preferred_element_type=jnp.float32)

    @pl.when(i > TC_POOL_START)
    def _accum_pool():
        ptc_ref[...] = ptc_ref[...] + jnp.dot(
            onehot, nfeats_ref[...], preferred_element_type=jnp.float32)


def _tc_mlp_body(pooled_ref, ptc_ref, counts_ref, emb_ref, w1_ref, b1_ref,
                 w2_ref, b2_ref, out_ref):
    e = emb_ref[...]
    pooled = jnp.sum(pooled_ref[...], axis=0) + ptc_ref[...]
    v = pooled + counts_ref[0][:, None] * e + e
    h = jnp.dot(v, w1_ref[...], preferred_element_type=jnp.float32)
    h = jnp.maximum(h + b1_ref[...], 0.0)
    o = jnp.dot(h, w2_ref[...], preferred_element_type=jnp.float32)
    out_ref[...] = o + b2_ref[...] + e


def kernel(nfeats, batch_id, init_emb, W1, b1, W2, b2):
    bid = batch_id.astype(jnp.int32)

    pooled_sc = _sc_segsum(nfeats, bid)

    bid3 = bid.reshape(N_TC_BLK, 1, R_BLK)
    nfeats_out, counts, pooled_tc = pl.pallas_call(
        _tc_add_body,
        grid=(N_TC_BLK,),
        in_specs=[
            pl.BlockSpec((1, 1, R_BLK), lambda i: (i, 0, 0)),
            pl.BlockSpec((R_BLK, D), lambda i: (i, 0)),
            pl.BlockSpec((1, D), lambda i: (0, 0)),
        ],
        out_specs=[
            pl.BlockSpec((R_BLK, D), lambda i: (i, 0)),
            pl.BlockSpec((1, B), lambda i: (0, 0)),
            pl.BlockSpec((B, D), lambda i: (0, 0)),
        ],
        out_shape=[
            jax.ShapeDtypeStruct((N, D), jnp.float32),
            jax.ShapeDtypeStruct((1, B), jnp.float32),
            jax.ShapeDtypeStruct((B, D), jnp.float32),
        ],
    )(bid3, nfeats, init_emb)

    vnfeat_out = pl.pallas_call(
        _tc_mlp_body,
        out_shape=jax.ShapeDtypeStruct((B, D), jnp.float32),
    )(pooled_sc, pooled_tc, counts, init_emb, W1, b1.reshape(1, H), W2,
      b2.reshape(1, D))

    return nfeats_out, vnfeat_out


# speedup vs baseline: 1.8761x; 1.0050x over previous
import functools

import jax
import jax.numpy as jnp
from jax import lax
from jax.experimental import pallas as pl
from jax.experimental.pallas import tpu as pltpu
from jax.experimental.pallas import tpu_sc as plsc

N = 50000
B = 64
D = 256
H = 512

N_SC = 18000

NC = 2
NS = 16
NW = NC * NS

C_BLK = 80
N_SC_BLK = N_SC // C_BLK
V_MAX = (N_SC_BLK + NW - 1) // NW


def _sc_segsum_body(nfeats_hbm, bid_hbm, pooled_hbm, buf0, buf1, idx0, idx1,
                    acc, sem0, sem1):
    cid = lax.axis_index("c")
    sid = lax.axis_index("s")
    wid = sid * NC + cid

    def zero_row(r, carry):
        for j in range(D // 16):
            acc[r, pl.ds(j * 16, 16)] = jnp.zeros((16,), jnp.float32)
        return carry

    lax.fori_loop(0, B, zero_row, 0)

    nv = (N_SC_BLK - wid + NW - 1) // NW

    def start(v, buf, idxv, sem):
        base = (wid + v * NW) * C_BLK
        pltpu.async_copy(nfeats_hbm.at[pl.ds(base, C_BLK)], buf, sem)
        pltpu.async_copy(bid_hbm.at[pl.ds(base, C_BLK)], idxv, sem)

    def wait(buf, idxv, sem):
        pltpu.make_async_copy(nfeats_hbm.at[pl.ds(0, C_BLK)], buf, sem).wait()
        pltpu.make_async_copy(bid_hbm.at[pl.ds(0, C_BLK)], idxv, sem).wait()

    def process(buf, idxv):
        def grp(g, carry2):
            bids = idxv[pl.ds(g * 16, 16)]
            for lane in range(16):
                b = bids[lane]
                r = g * 16 + lane
                for j in range(D // 16):
                    plsc.addupdate(acc.at[b, pl.ds(j * 16, 16)],
                                   buf[r, pl.ds(j * 16, 16)])
            return carry2

        lax.fori_loop(0, C_BLK // 16, grp, 0)

    start(0, buf0, idx0, sem0)

    def pair(k, carry):
        i = 2 * k

        @pl.when(i + 1 < nv)
        def _start_odd():
            start(i + 1, buf1, idx1, sem1)

        @pl.when(i < nv)
        def _process_even():
            wait(buf0, idx0, sem0)
            process(buf0, idx0)

        @pl.when(i + 2 < nv)
        def _start_even():
            start(i + 2, buf0, idx0, sem0)

        @pl.when(i + 1 < nv)
        def _process_odd():
            wait(buf1, idx1, sem1)
            process(buf1, idx1)

        return carry

    lax.fori_loop(0, (V_MAX + 1) // 2, pair, 0)

    pltpu.sync_copy(acc, pooled_hbm.at[wid])


_sc_segsum = functools.partial(
    pl.kernel,
    out_type=jax.ShapeDtypeStruct((NW, B, D), jnp.float32),
    mesh=plsc.VectorSubcoreMesh(
        core_axis_name="c", subcore_axis_name="s",
        num_cores=NC, num_subcores=NS,
    ),
    scratch_types=[
        pltpu.VMEM((C_BLK, D), jnp.float32),
        pltpu.VMEM((C_BLK, D), jnp.float32),
        pltpu.VMEM((C_BLK,), jnp.int32),
        pltpu.VMEM((C_BLK,), jnp.int32),
        pltpu.VMEM((B, D), jnp.float32),
        pltpu.SemaphoreType.DMA,
        pltpu.SemaphoreType.DMA,
    ],
)(_sc_segsum_body)


R_BLK = 2000
N_TC_BLK = N // R_BLK
TC_POOL_START = N_SC // R_BLK


def _tc_add_body(bid_ref, nfeats_ref, emb_ref, out_ref, counts_ref, ptc_ref):
    i = pl.program_id(0)
    out_ref[...] = nfeats_ref[...] + emb_ref[...]
    ids = bid_ref[0]
    g = lax.broadcasted_iota(jnp.int32, (B, R_BLK), 0)
    onehot = (ids == g).astype(jnp.float32)
    c = jnp.sum(onehot, axis=1)

    @pl.when(i == 0)
    def _init_counts():
        counts_ref[...] = c[None, :]

    @pl.when(i > 0)
    def _accum_counts():
        counts_ref[...] = counts_ref[...] + c[None, :]

    @pl.when(i == TC_POOL_START)
    def _init_pool():
        ptc_ref[...] = jnp.dot(onehot, nfeats_ref[...],
                               preferred_element_type=jnp.float32)

    @pl.when(i > TC_POOL_START)
    def _accum_pool():
        ptc_ref[...] = ptc_ref[...] + jnp.dot(
            onehot, nfeats_ref[...], preferred_element_type=jnp.float32)


def _tc_mlp_body(pooled_ref, ptc_ref, counts_ref, emb_ref, w1_ref, b1_ref,
                 w2_ref, b2_ref, out_ref):
    e = emb_ref[...]
    pooled = jnp.sum(pooled_ref[...], axis=0) + ptc_ref[...]
    v = pooled + counts_ref[0][:, None] * e + e
    h = jnp.dot(v, w1_ref[...], preferred_element_type=jnp.float32)
    h = jnp.maximum(h + b1_ref[...], 0.0)
    o = jnp.dot(h, w2_ref[...], preferred_element_type=jnp.float32)
    out_ref[...] = o + b2_ref[...] + e


def kernel(nfeats, batch_id, init_emb, W1, b1, W2, b2):
    bid = batch_id.astype(jnp.int32)

    pooled_sc = _sc_segsum(nfeats, bid)

    bid3 = bid.reshape(N_TC_BLK, 1, R_BLK)
    nfeats_out, counts, pooled_tc = pl.pallas_call(
        _tc_add_body,
        grid=(N_TC_BLK,),
        in_specs=[
            pl.BlockSpec((1, 1, R_BLK), lambda i: (i, 0, 0)),
            pl.BlockSpec((R_BLK, D), lambda i: (i, 0)),
            pl.BlockSpec((1, D), lambda i: (0, 0)),
        ],
        out_specs=[
            pl.BlockSpec((R_BLK, D), lambda i: (i, 0)),
            pl.BlockSpec((1, B), lambda i: (0, 0)),
            pl.BlockSpec((B, D), lambda i: (0, 0)),
        ],
        out_shape=[
            jax.ShapeDtypeStruct((N, D), jnp.float32),
            jax.ShapeDtypeStruct((1, B), jnp.float32),
            jax.ShapeDtypeStruct((B, D), jnp.float32),
        ],
    )(bid3, nfeats, init_emb)

    vnfeat_out = pl.pallas_call(
        _tc_mlp_body,
        out_shape=jax.ShapeDtypeStruct((B, D), jnp.float32),
    )(pooled_sc, pooled_tc, counts, init_emb, W1, b1.reshape(1, H), W2,
      b2.reshape(1, D))

    return nfeats_out, vnfeat_out


# speedup vs baseline: 1.9803x; 1.0555x over previous
import functools

import jax
import jax.numpy as jnp
from jax import lax
from jax.experimental import pallas as pl
from jax.experimental.pallas import tpu as pltpu
from jax.experimental.pallas import tpu_sc as plsc

N = 50000
B = 64
D = 256
H = 512

N_SC = 16000

NC = 2
NS = 16
NW = NC * NS

C_BLK = 80
N_SC_BLK = N_SC // C_BLK
V_MAX = (N_SC_BLK + NW - 1) // NW


def _sc_segsum_body(nfeats_hbm, bid_hbm, pooled_hbm, buf0, buf1, idx0, idx1,
                    acc, sem0, sem1):
    cid = lax.axis_index("c")
    sid = lax.axis_index("s")
    wid = sid * NC + cid

    def zero_row(r, carry):
        for j in range(D // 16):
            acc[r, pl.ds(j * 16, 16)] = jnp.zeros((16,), jnp.float32)
        return carry

    lax.fori_loop(0, B, zero_row, 0)

    nv = (N_SC_BLK - wid + NW - 1) // NW

    def start(v, buf, idxv, sem):
        base = (wid + v * NW) * C_BLK
        pltpu.async_copy(nfeats_hbm.at[pl.ds(base, C_BLK)], buf, sem)
        pltpu.async_copy(bid_hbm.at[pl.ds(base, C_BLK)], idxv, sem)

    def wait(buf, idxv, sem):
        pltpu.make_async_copy(nfeats_hbm.at[pl.ds(0, C_BLK)], buf, sem).wait()
        pltpu.make_async_copy(bid_hbm.at[pl.ds(0, C_BLK)], idxv, sem).wait()

    def process(buf, idxv):
        def grp(g, carry2):
            bids = idxv[pl.ds(g * 16, 16)]
            for lane in range(16):
                b = bids[lane]
                r = g * 16 + lane
                for j in range(D // 16):
                    plsc.addupdate(acc.at[b, pl.ds(j * 16, 16)],
                                   buf[r, pl.ds(j * 16, 16)])
            return carry2

        lax.fori_loop(0, C_BLK // 16, grp, 0)

    start(0, buf0, idx0, sem0)

    def pair(k, carry):
        i = 2 * k

        @pl.when(i + 1 < nv)
        def _start_odd():
            start(i + 1, buf1, idx1, sem1)

        @pl.when(i < nv)
        def _process_even():
            wait(buf0, idx0, sem0)
            process(buf0, idx0)

        @pl.when(i + 2 < nv)
        def _start_even():
            start(i + 2, buf0, idx0, sem0)

        @pl.when(i + 1 < nv)
        def _process_odd():
            wait(buf1, idx1, sem1)
            process(buf1, idx1)

        return carry

    lax.fori_loop(0, (V_MAX + 1) // 2, pair, 0)

    pltpu.sync_copy(acc, pooled_hbm.at[wid])


_sc_segsum = functools.partial(
    pl.kernel,
    out_type=jax.ShapeDtypeStruct((NW, B, D), jnp.float32),
    mesh=plsc.VectorSubcoreMesh(
        core_axis_name="c", subcore_axis_name="s",
        num_cores=NC, num_subcores=NS,
    ),
    scratch_types=[
        pltpu.VMEM((C_BLK, D), jnp.float32),
        pltpu.VMEM((C_BLK, D), jnp.float32),
        pltpu.VMEM((C_BLK,), jnp.int32),
        pltpu.VMEM((C_BLK,), jnp.int32),
        pltpu.VMEM((B, D), jnp.float32),
        pltpu.SemaphoreType.DMA,
        pltpu.SemaphoreType.DMA,
    ],
)(_sc_segsum_body)


R_BLK = 2000
N_TC_BLK = N // R_BLK
TC_POOL_START = N_SC // R_BLK


def _tc_add_body(bid_ref, nfeats_ref, emb_ref, out_ref, counts_ref, ptc_ref):
    i = pl.program_id(0)
    out_ref[...] = nfeats_ref[...] + emb_ref[...]
    ids = bid_ref[0]
    g = lax.broadcasted_iota(jnp.int32, (B, R_BLK), 0)
    onehot = (ids == g).astype(jnp.float32)
    c = jnp.sum(onehot, axis=1)

    @pl.when(i == 0)
    def _init_counts():
        counts_ref[...] = c[None, :]

    @pl.when(i > 0)
    def _accum_counts():
        counts_ref[...] = counts_ref[...] + c[None, :]

    @pl.when(i == TC_POOL_START)
    def _init_pool():
        ptc_ref[...] = jnp.dot(onehot, nfeats_ref[...],
                               preferred_element_type=jnp.float32)

    @pl.when(i > TC_POOL_START)
    def _accum_pool():
        ptc_ref[...] = ptc_ref[...] + jnp.dot(
            onehot, nfeats_ref[...], preferred_element_type=jnp.float32)


def _tc_mlp_body(pooled_ref, ptc_ref, counts_ref, emb_ref, w1_ref, b1_ref,
                 w2_ref, b2_ref, out_ref):
    e = emb_ref[...]
    pooled = jnp.sum(pooled_ref[...], axis=0) + ptc_ref[...]
    v = pooled + counts_ref[0][:, None] * e + e
    h = jnp.dot(v, w1_ref[...], preferred_element_type=jnp.float32)
    h = jnp.maximum(h + b1_ref[...], 0.0)
    o = jnp.dot(h, w2_ref[...], preferred_element_type=jnp.float32)
    out_ref[...] = o + b2_ref[...] + e


def kernel(nfeats, batch_id, init_emb, W1, b1, W2, b2):
    bid = batch_id.astype(jnp.int32)

    pooled_sc = _sc_segsum(nfeats, bid)

    bid3 = bid.reshape(N_TC_BLK, 1, R_BLK)
    nfeats_out, counts, pooled_tc = pl.pallas_call(
        _tc_add_body,
        grid=(N_TC_BLK,),
        in_specs=[
            pl.BlockSpec((1, 1, R_BLK), lambda i: (i, 0, 0)),
            pl.BlockSpec((R_BLK, D), lambda i: (i, 0)),
            pl.BlockSpec((1, D), lambda i: (0, 0)),
        ],
        out_specs=[
            pl.BlockSpec((R_BLK, D), lambda i: (i, 0)),
            pl.BlockSpec((1, B), lambda i: (0, 0)),
            pl.BlockSpec((B, D), lambda i: (0, 0)),
        ],
        out_shape=[
            jax.ShapeDtypeStruct((N, D), jnp.float32),
            jax.ShapeDtypeStruct((1, B), jnp.float32),
            jax.ShapeDtypeStruct((B, D), jnp.float32),
        ],
    )(bid3, nfeats, init_emb)

    vnfeat_out = pl.pallas_call(
        _tc_mlp_body,
        out_shape=jax.ShapeDtypeStruct((B, D), jnp.float32),
    )(pooled_sc, pooled_tc, counts, init_emb, W1, b1.reshape(1, H), W2,
      b2.reshape(1, D))

    return nfeats_out, vnfeat_out


# speedup vs baseline: 1.9960x; 1.0079x over previous
import functools

import jax
import jax.numpy as jnp
from jax import lax
from jax.experimental import pallas as pl
from jax.experimental.pallas import tpu as pltpu
from jax.experimental.pallas import tpu_sc as plsc

N = 50000
B = 64
D = 256
H = 512

N_SC = 12000

NC = 2
NS = 16
NW = NC * NS

C_BLK = 80
N_SC_BLK = N_SC // C_BLK
V_MAX = (N_SC_BLK + NW - 1) // NW


def _sc_segsum_body(nfeats_hbm, bid_hbm, pooled_hbm, buf0, buf1, idx0, idx1,
                    acc, sem0, sem1):
    cid = lax.axis_index("c")
    sid = lax.axis_index("s")
    wid = sid * NC + cid

    def zero_row(r, carry):
        for j in range(D // 16):
            acc[r, pl.ds(j * 16, 16)] = jnp.zeros((16,), jnp.float32)
        return carry

    lax.fori_loop(0, B, zero_row, 0)

    nv = (N_SC_BLK - wid + NW - 1) // NW

    def start(v, buf, idxv, sem):
        base = (wid + v * NW) * C_BLK
        pltpu.async_copy(nfeats_hbm.at[pl.ds(base, C_BLK)], buf, sem)
        pltpu.async_copy(bid_hbm.at[pl.ds(base, C_BLK)], idxv, sem)

    def wait(buf, idxv, sem):
        pltpu.make_async_copy(nfeats_hbm.at[pl.ds(0, C_BLK)], buf, sem).wait()
        pltpu.make_async_copy(bid_hbm.at[pl.ds(0, C_BLK)], idxv, sem).wait()

    def process(buf, idxv):
        def grp(g, carry2):
            bids = idxv[pl.ds(g * 16, 16)]
            for lane in range(16):
                b = bids[lane]
                r = g * 16 + lane
                for j in range(D // 16):
                    plsc.addupdate(acc.at[b, pl.ds(j * 16, 16)],
                                   buf[r, pl.ds(j * 16, 16)])
            return carry2

        lax.fori_loop(0, C_BLK // 16, grp, 0)

    start(0, buf0, idx0, sem0)

    def pair(k, carry):
        i = 2 * k

        @pl.when(i + 1 < nv)
        def _start_odd():
            start(i + 1, buf1, idx1, sem1)

        @pl.when(i < nv)
        def _process_even():
            wait(buf0, idx0, sem0)
            process(buf0, idx0)

        @pl.when(i + 2 < nv)
        def _start_even():
            start(i + 2, buf0, idx0, sem0)

        @pl.when(i + 1 < nv)
        def _process_odd():
            wait(buf1, idx1, sem1)
            process(buf1, idx1)

        return carry

    lax.fori_loop(0, (V_MAX + 1) // 2, pair, 0)

    pltpu.sync_copy(acc, pooled_hbm.at[wid])


_sc_segsum = functools.partial(
    pl.kernel,
    out_type=jax.ShapeDtypeStruct((NW, B, D), jnp.float32),
    mesh=plsc.VectorSubcoreMesh(
        core_axis_name="c", subcore_axis_name="s",
        num_cores=NC, num_subcores=NS,
    ),
    scratch_types=[
        pltpu.VMEM((C_BLK, D), jnp.float32),
        pltpu.VMEM((C_BLK, D), jnp.float32),
        pltpu.VMEM((C_BLK,), jnp.int32),
        pltpu.VMEM((C_BLK,), jnp.int32),
        pltpu.VMEM((B, D), jnp.float32),
        pltpu.SemaphoreType.DMA,
        pltpu.SemaphoreType.DMA,
    ],
)(_sc_segsum_body)


R_BLK = 2000
N_TC_BLK = N // R_BLK
TC_POOL_START = N_SC // R_BLK


def _tc_add_body(bid_ref, nfeats_ref, emb_ref, out_ref, counts_ref, ptc_ref):
    i = pl.program_id(0)
    out_ref[...] = nfeats_ref[...] + emb_ref[...]
    ids = bid_ref[0]
    g = lax.broadcasted_iota(jnp.int32, (B, R_BLK), 0)
    onehot = (ids == g).astype(jnp.float32)
    c = jnp.sum(onehot, axis=1)

    @pl.when(i == 0)
    def _init_counts():
        counts_ref[...] = c[None, :]

    @pl.when(i > 0)
    def _accum_counts():
        counts_ref[...] = counts_ref[...] + c[None, :]

    @pl.when(i == TC_POOL_START)
    def _init_pool():
        ptc_ref[...] = jnp.dot(onehot, nfeats_ref[...],
                               preferred_element_type=jnp.float32)

    @pl.when(i > TC_POOL_START)
    def _accum_pool():
        ptc_ref[...] = ptc_ref[...] + jnp.dot(
            onehot, nfeats_ref[...], preferred_element_type=jnp.float32)


def _tc_mlp_body(pooled_ref, ptc_ref, counts_ref, emb_ref, w1_ref, b1_ref,
                 w2_ref, b2_ref, out_ref):
    e = emb_ref[...]
    pooled = jnp.sum(pooled_ref[...], axis=0) + ptc_ref[...]
    v = pooled + counts_ref[0][:, None] * e + e
    h = jnp.dot(v, w1_ref[...], preferred_element_type=jnp.float32)
    h = jnp.maximum(h + b1_ref[...], 0.0)
    o = jnp.dot(h, w2_ref[...], preferred_element_type=jnp.float32)
    out_ref[...] = o + b2_ref[...] + e


def kernel(nfeats, batch_id, init_emb, W1, b1, W2, b2):
    bid = batch_id.astype(jnp.int32)

    pooled_sc = _sc_segsum(nfeats, bid)

    bid3 = bid.reshape(N_TC_BLK, 1, R_BLK)
    nfeats_out, counts, pooled_tc = pl.pallas_call(
        _tc_add_body,
        grid=(N_TC_BLK,),
        in_specs=[
            pl.BlockSpec((1, 1, R_BLK), lambda i: (i, 0, 0)),
            pl.BlockSpec((R_BLK, D), lambda i: (i, 0)),
            pl.BlockSpec((1, D), lambda i: (0, 0)),
        ],
        out_specs=[
            pl.BlockSpec((R_BLK, D), lambda i: (i, 0)),
            pl.BlockSpec((1, B), lambda i: (0, 0)),
            pl.BlockSpec((B, D), lambda i: (0, 0)),
        ],
        out_shape=[
            jax.ShapeDtypeStruct((N, D), jnp.float32),
            jax.ShapeDtypeStruct((1, B), jnp.float32),
            jax.ShapeDtypeStruct((B, D), jnp.float32),
        ],
    )(bid3, nfeats, init_emb)

    vnfeat_out = pl.pallas_call(
        _tc_mlp_body,
        out_shape=jax.ShapeDtypeStruct((B, D), jnp.float32),
    )(pooled_sc, pooled_tc, counts, init_emb, W1, b1.reshape(1, H), W2,
      b2.reshape(1, D))

    return nfeats_out, vnfeat_out


# speedup vs baseline: 1.9993x; 1.0017x over previous
import functools

import jax
import jax.numpy as jnp
from jax import lax
from jax.experimental import pallas as pl
from jax.experimental.pallas import tpu as pltpu
from jax.experimental.pallas import tpu_sc as plsc

N = 50000
B = 64
D = 256
H = 512

N_SC = 10000

NC = 2
NS = 16
NW = NC * NS

C_BLK = 80
N_SC_BLK = N_SC // C_BLK
V_MAX = (N_SC_BLK + NW - 1) // NW


def _sc_segsum_body(nfeats_hbm, bid_hbm, pooled_hbm, buf0, buf1, idx0, idx1,
                    acc, sem0, sem1):
    cid = lax.axis_index("c")
    sid = lax.axis_index("s")
    wid = sid * NC + cid

    def zero_row(r, carry):
        for j in range(D // 16):
            acc[r, pl.ds(j * 16, 16)] = jnp.zeros((16,), jnp.float32)
        return carry

    lax.fori_loop(0, B, zero_row, 0)

    nv = (N_SC_BLK - wid + NW - 1) // NW

    def start(v, buf, idxv, sem):
        base = (wid + v * NW) * C_BLK
        pltpu.async_copy(nfeats_hbm.at[pl.ds(base, C_BLK)], buf, sem)
        pltpu.async_copy(bid_hbm.at[pl.ds(base, C_BLK)], idxv, sem)

    def wait(buf, idxv, sem):
        pltpu.make_async_copy(nfeats_hbm.at[pl.ds(0, C_BLK)], buf, sem).wait()
        pltpu.make_async_copy(bid_hbm.at[pl.ds(0, C_BLK)], idxv, sem).wait()

    def process(buf, idxv):
        def grp(g, carry2):
            bids = idxv[pl.ds(g * 16, 16)]
            for lane in range(16):
                b = bids[lane]
                r = g * 16 + lane
                for j in range(D // 16):
                    plsc.addupdate(acc.at[b, pl.ds(j * 16, 16)],
                                   buf[r, pl.ds(j * 16, 16)])
            return carry2

        lax.fori_loop(0, C_BLK // 16, grp, 0)

    start(0, buf0, idx0, sem0)

    def pair(k, carry):
        i = 2 * k

        @pl.when(i + 1 < nv)
        def _start_odd():
            start(i + 1, buf1, idx1, sem1)

        @pl.when(i < nv)
        def _process_even():
            wait(buf0, idx0, sem0)
            process(buf0, idx0)

        @pl.when(i + 2 < nv)
        def _start_even():
            start(i + 2, buf0, idx0, sem0)

        @pl.when(i + 1 < nv)
        def _process_odd():
            wait(buf1, idx1, sem1)
            process(buf1, idx1)

        return carry

    lax.fori_loop(0, (V_MAX + 1) // 2, pair, 0)

    pltpu.sync_copy(acc, pooled_hbm.at[wid])


_sc_segsum = functools.partial(
    pl.kernel,
    out_type=jax.ShapeDtypeStruct((NW, B, D), jnp.float32),
    mesh=plsc.VectorSubcoreMesh(
        core_axis_name="c", subcore_axis_name="s",
        num_cores=NC, num_subcores=NS,
    ),
    scratch_types=[
        pltpu.VMEM((C_BLK, D), jnp.float32),
        pltpu.VMEM((C_BLK, D), jnp.float32),
        pltpu.VMEM((C_BLK,), jnp.int32),
        pltpu.VMEM((C_BLK,), jnp.int32),
        pltpu.VMEM((B, D), jnp.float32),
        pltpu.SemaphoreType.DMA,
        pltpu.SemaphoreType.DMA,
    ],
)(_sc_segsum_body)


R_BLK = 2000
N_TC_BLK = N // R_BLK
TC_POOL_START = N_SC // R_BLK


def _tc_add_body(bid_ref, nfeats_ref, emb_ref, out_ref, counts_ref, ptc_ref):
    i = pl.program_id(0)
    out_ref[...] = nfeats_ref[...] + emb_ref[...]
    ids = bid_ref[0]
    g = lax.broadcasted_iota(jnp.int32, (B, R_BLK), 0)
    onehot = (ids == g).astype(jnp.float32)
    c = jnp.sum(onehot, axis=1)

    @pl.when(i == 0)
    def _init_counts():
        counts_ref[...] = c[None, :]

    @pl.when(i > 0)
    def _accum_counts():
        counts_ref[...] = counts_ref[...] + c[None, :]

    @pl.when(i == TC_POOL_START)
    def _init_pool():
        ptc_ref[...] = jnp.dot(onehot, nfeats_ref[...],
                               preferred_element_type=jnp.float32)

    @pl.when(i > TC_POOL_START)
    def _accum_pool():
        ptc_ref[...] = ptc_ref[...] + jnp.dot(
            onehot, nfeats_ref[...], preferred_element_type=jnp.float32)


def _tc_mlp_body(pooled_ref, ptc_ref, counts_ref, emb_ref, w1_ref, b1_ref,
                 w2_ref, b2_ref, out_ref):
    e = emb_ref[...]
    pooled = jnp.sum(pooled_ref[...], axis=0) + ptc_ref[...]
    v = pooled + counts_ref[0][:, None] * e + e
    h = jnp.dot(v, w1_ref[...], preferred_element_type=jnp.float32)
    h = jnp.maximum(h + b1_ref[...], 0.0)
    o = jnp.dot(h, w2_ref[...], preferred_element_type=jnp.float32)
    out_ref[...] = o + b2_ref[...] + e


def kernel(nfeats, batch_id, init_emb, W1, b1, W2, b2):
    bid = batch_id.astype(jnp.int32)

    pooled_sc = _sc_segsum(nfeats, bid)

    bid3 = bid.reshape(N_TC_BLK, 1, R_BLK)
    nfeats_out, counts, pooled_tc = pl.pallas_call(
        _tc_add_body,
        grid=(N_TC_BLK,),
        in_specs=[
            pl.BlockSpec((1, 1, R_BLK), lambda i: (i, 0, 0)),
            pl.BlockSpec((R_BLK, D), lambda i: (i, 0)),
            pl.BlockSpec((1, D), lambda i: (0, 0)),
        ],
        out_specs=[
            pl.BlockSpec((R_BLK, D), lambda i: (i, 0)),
            pl.BlockSpec((1, B), lambda i: (0, 0)),
            pl.BlockSpec((B, D), lambda i: (0, 0)),
        ],
        out_shape=[
            jax.ShapeDtypeStruct((N, D), jnp.float32),
            jax.ShapeDtypeStruct((1, B), jnp.float32),
            jax.ShapeDtypeStruct((B, D), jnp.float32),
        ],
    )(bid3, nfeats, init_emb)

    vnfeat_out = pl.pallas_call(
        _tc_mlp_body,
        out_shape=jax.ShapeDtypeStruct((B, D), jnp.float32),
    )(pooled_sc, pooled_tc, counts, init_emb, W1, b1.reshape(1, H), W2,
      b2.reshape(1, D))

    return nfeats_out, vnfeat_out


# speedup vs baseline: 2.1555x; 1.0781x over previous
import functools

import jax
import jax.numpy as jnp
from jax import lax
from jax.experimental import pallas as pl
from jax.experimental.pallas import tpu as pltpu
from jax.experimental.pallas import tpu_sc as plsc

N = 50000
B = 64
D = 256
H = 512

N_SC = 10000

NC = 2
NS = 16
NW = NC * NS

C_BLK = 80
N_SC_BLK = N_SC // C_BLK
V_MAX = (N_SC_BLK + NW - 1) // NW


def _sc_segsum_body(nfeats_hbm, bid_hbm, pooled_hbm, buf0, buf1, idx0, idx1,
                    acc, sem0, sem1):
    cid = lax.axis_index("c")
    sid = lax.axis_index("s")
    wid = sid * NC + cid

    def zero_row(r, carry):
        for j in range(D // 16):
            acc[r, pl.ds(j * 16, 16)] = jnp.zeros((16,), jnp.float32)
        return carry

    lax.fori_loop(0, B, zero_row, 0)

    nv = (N_SC_BLK - wid + NW - 1) // NW

    def start(v, buf, idxv, sem):
        base = (wid + v * NW) * C_BLK
        pltpu.async_copy(nfeats_hbm.at[pl.ds(base, C_BLK)], buf, sem)
        pltpu.async_copy(bid_hbm.at[pl.ds(base, C_BLK)], idxv, sem)

    def wait(buf, idxv, sem):
        pltpu.make_async_copy(nfeats_hbm.at[pl.ds(0, C_BLK)], buf, sem).wait()
        pltpu.make_async_copy(bid_hbm.at[pl.ds(0, C_BLK)], idxv, sem).wait()

    def process(buf, idxv):
        def grp(g, carry2):
            bids = idxv[pl.ds(g * 16, 16)]
            for lane in range(16):
                b = bids[lane]
                r = g * 16 + lane
                for j in range(D // 16):
                    plsc.addupdate(acc.at[b, pl.ds(j * 16, 16)],
                                   buf[r, pl.ds(j * 16, 16)])
            return carry2

        lax.fori_loop(0, C_BLK // 16, grp, 0)

    start(0, buf0, idx0, sem0)

    def pair(k, carry):
        i = 2 * k

        @pl.when(i + 1 < nv)
        def _start_odd():
            start(i + 1, buf1, idx1, sem1)

        @pl.when(i < nv)
        def _process_even():
            wait(buf0, idx0, sem0)
            process(buf0, idx0)

        @pl.when(i + 2 < nv)
        def _start_even():
            start(i + 2, buf0, idx0, sem0)

        @pl.when(i + 1 < nv)
        def _process_odd():
            wait(buf1, idx1, sem1)
            process(buf1, idx1)

        return carry

    lax.fori_loop(0, (V_MAX + 1) // 2, pair, 0)

    pltpu.sync_copy(acc, pooled_hbm.at[wid])


_sc_segsum = functools.partial(
    pl.kernel,
    out_type=jax.ShapeDtypeStruct((NW, B, D), jnp.float32),
    mesh=plsc.VectorSubcoreMesh(
        core_axis_name="c", subcore_axis_name="s",
        num_cores=NC, num_subcores=NS,
    ),
    scratch_types=[
        pltpu.VMEM((C_BLK, D), jnp.float32),
        pltpu.VMEM((C_BLK, D), jnp.float32),
        pltpu.VMEM((C_BLK,), jnp.int32),
        pltpu.VMEM((C_BLK,), jnp.int32),
        pltpu.VMEM((B, D), jnp.float32),
        pltpu.SemaphoreType.DMA,
        pltpu.SemaphoreType.DMA,
    ],
)(_sc_segsum_body)


R_BLK = 5000
N_TC_BLK = N // R_BLK
TC_POOL_START = N_SC // R_BLK


def _tc_add_body(bid_ref, nfeats_ref, emb_ref, out_ref, counts_ref, ptc_ref):
    i = pl.program_id(0)
    out_ref[...] = nfeats_ref[...] + emb_ref[...]
    ids = bid_ref[0]
    g = lax.broadcasted_iota(jnp.int32, (B, R_BLK), 0)
    onehot = (ids == g).astype(jnp.float32)
    c = jnp.sum(onehot, axis=1)

    @pl.when(i == 0)
    def _init_counts():
        counts_ref[...] = c[None, :]

    @pl.when(i > 0)
    def _accum_counts():
        counts_ref[...] = counts_ref[...] + c[None, :]

    @pl.when(i == TC_POOL_START)
    def _init_pool():
        ptc_ref[...] = jnp.dot(onehot, nfeats_ref[...],
                               preferred_element_type=jnp.float32)

    @pl.when(i > TC_POOL_START)
    def _accum_pool():
        ptc_ref[...] = ptc_ref[...] + jnp.dot(
            onehot, nfeats_ref[...], preferred_element_type=jnp.float32)


def _tc_mlp_body(pooled_ref, ptc_ref, counts_ref, emb_ref, w1_ref, b1_ref,
                 w2_ref, b2_ref, out_ref):
    e = emb_ref[...]
    pooled = jnp.sum(pooled_ref[...], axis=0) + ptc_ref[...]
    v = pooled + counts_ref[0][:, None] * e + e
    h = jnp.dot(v, w1_ref[...], preferred_element_type=jnp.float32)
    h = jnp.maximum(h + b1_ref[...], 0.0)
    o = jnp.dot(h, w2_ref[...], preferred_element_type=jnp.float32)
    out_ref[...] = o + b2_ref[...] + e


def kernel(nfeats, batch_id, init_emb, W1, b1, W2, b2):
    bid = batch_id.astype(jnp.int32)

    pooled_sc = _sc_segsum(nfeats, bid)

    bid3 = bid.reshape(N_TC_BLK, 1, R_BLK)
    nfeats_out, counts, pooled_tc = pl.pallas_call(
        _tc_add_body,
        grid=(N_TC_BLK,),
        in_specs=[
            pl.BlockSpec((1, 1, R_BLK), lambda i: (i, 0, 0)),
            pl.BlockSpec((R_BLK, D), lambda i: (i, 0)),
            pl.BlockSpec((1, D), lambda i: (0, 0)),
        ],
        out_specs=[
            pl.BlockSpec((R_BLK, D), lambda i: (i, 0)),
            pl.BlockSpec((1, B), lambda i: (0, 0)),
            pl.BlockSpec((B, D), lambda i: (0, 0)),
        ],
        out_shape=[
            jax.ShapeDtypeStruct((N, D), jnp.float32),
            jax.ShapeDtypeStruct((1, B), jnp.float32),
            jax.ShapeDtypeStruct((B, D), jnp.float32),
        ],
    )(bid3, nfeats, init_emb)

    vnfeat_out = pl.pallas_call(
        _tc_mlp_body,
        out_shape=jax.ShapeDtypeStruct((B, D), jnp.float32),
    )(pooled_sc, pooled_tc, counts, init_emb, W1, b1.reshape(1, H), W2,
      b2.reshape(1, D))

    return nfeats_out, vnfeat_out


# speedup vs baseline: 2.2142x; 1.0272x over previous
import functools

import jax
import jax.numpy as jnp
from jax import lax
from jax.experimental import pallas as pl
from jax.experimental.pallas import tpu as pltpu
from jax.experimental.pallas import tpu_sc as plsc

N = 50000
B = 64
D = 256
H = 512

N_SC = 10000

NC = 2
NS = 16
NW = NC * NS

C_BLK = 80
N_SC_BLK = N_SC // C_BLK
V_MAX = (N_SC_BLK + NW - 1) // NW


def _sc_segsum_body(nfeats_hbm, bid_hbm, pooled_hbm, buf0, buf1, idx0, idx1,
                    acc, sem0, sem1):
    cid = lax.axis_index("c")
    sid = lax.axis_index("s")
    wid = sid * NC + cid

    def zero_row(r, carry):
        for j in range(D // 16):
            acc[r, pl.ds(j * 16, 16)] = jnp.zeros((16,), jnp.float32)
        return carry

    lax.fori_loop(0, B, zero_row, 0)

    nv = (N_SC_BLK - wid + NW - 1) // NW

    def start(v, buf, idxv, sem):
        base = (wid + v * NW) * C_BLK
        pltpu.async_copy(nfeats_hbm.at[pl.ds(base, C_BLK)], buf, sem)
        pltpu.async_copy(bid_hbm.at[pl.ds(base, C_BLK)], idxv, sem)

    def wait(buf, idxv, sem):
        pltpu.make_async_copy(nfeats_hbm.at[pl.ds(0, C_BLK)], buf, sem).wait()
        pltpu.make_async_copy(bid_hbm.at[pl.ds(0, C_BLK)], idxv, sem).wait()

    def process(buf, idxv):
        def grp(g, carry2):
            bids = idxv[pl.ds(g * 16, 16)]
            for lane in range(16):
                b = bids[lane]
                r = g * 16 + lane
                for j in range(D // 16):
                    plsc.addupdate(acc.at[b, pl.ds(j * 16, 16)],
                                   buf[r, pl.ds(j * 16, 16)])
            return carry2

        lax.fori_loop(0, C_BLK // 16, grp, 0)

    start(0, buf0, idx0, sem0)

    def pair(k, carry):
        i = 2 * k

        @pl.when(i + 1 < nv)
        def _start_odd():
            start(i + 1, buf1, idx1, sem1)

        @pl.when(i < nv)
        def _process_even():
            wait(buf0, idx0, sem0)
            process(buf0, idx0)

        @pl.when(i + 2 < nv)
        def _start_even():
            start(i + 2, buf0, idx0, sem0)

        @pl.when(i + 1 < nv)
        def _process_odd():
            wait(buf1, idx1, sem1)
            process(buf1, idx1)

        return carry

    lax.fori_loop(0, (V_MAX + 1) // 2, pair, 0)

    pltpu.sync_copy(acc, pooled_hbm.at[wid])


_sc_segsum = functools.partial(
    pl.kernel,
    out_type=jax.ShapeDtypeStruct((NW, B, D), jnp.float32),
    mesh=plsc.VectorSubcoreMesh(
        core_axis_name="c", subcore_axis_name="s",
        num_cores=NC, num_subcores=NS,
    ),
    scratch_types=[
        pltpu.VMEM((C_BLK, D), jnp.float32),
        pltpu.VMEM((C_BLK, D), jnp.float32),
        pltpu.VMEM((C_BLK,), jnp.int32),
        pltpu.VMEM((C_BLK,), jnp.int32),
        pltpu.VMEM((B, D), jnp.float32),
        pltpu.SemaphoreType.DMA,
        pltpu.SemaphoreType.DMA,
    ],
)(_sc_segsum_body)


R_BLK = 10000
N_TC_BLK = N // R_BLK
TC_POOL_START = N_SC // R_BLK


def _tc_add_body(bid_ref, nfeats_ref, emb_ref, out_ref, counts_ref, ptc_ref):
    i = pl.program_id(0)
    out_ref[...] = nfeats_ref[...] + emb_ref[...]
    ids = bid_ref[0]
    g = lax.broadcasted_iota(jnp.int32, (B, R_BLK), 0)
    onehot = (ids == g).astype(jnp.float32)
    c = jnp.sum(onehot, axis=1)

    @pl.when(i == 0)
    def _init_counts():
        counts_ref[...] = c[None, :]

    @pl.when(i > 0)
    def _accum_counts():
        counts_ref[...] = counts_ref[...] + c[None, :]

    @pl.when(i == TC_POOL_START)
    def _init_pool():
        ptc_ref[...] = jnp.dot(onehot, nfeats_ref[...],
                               preferred_element_type=jnp.float32)

    @pl.when(i > TC_POOL_START)
    def _accum_pool():
        ptc_ref[...] = ptc_ref[...] + jnp.dot(
            onehot, nfeats_ref[...], preferred_element_type=jnp.float32)


def _tc_mlp_body(pooled_ref, ptc_ref, counts_ref, emb_ref, w1_ref, b1_ref,
                 w2_ref, b2_ref, out_ref):
    e = emb_ref[...]
    pooled = jnp.sum(pooled_ref[...], axis=0) + ptc_ref[...]
    v = pooled + counts_ref[0][:, None] * e + e
    h = jnp.dot(v, w1_ref[...], preferred_element_type=jnp.float32)
    h = jnp.maximum(h + b1_ref[...], 0.0)
    o = jnp.dot(h, w2_ref[...], preferred_element_type=jnp.float32)
    out_ref[...] = o + b2_ref[...] + e


def kernel(nfeats, batch_id, init_emb, W1, b1, W2, b2):
    bid = batch_id.astype(jnp.int32)

    pooled_sc = _sc_segsum(nfeats, bid)

    bid3 = bid.reshape(N_TC_BLK, 1, R_BLK)
    nfeats_out, counts, pooled_tc = pl.pallas_call(
        _tc_add_body,
        grid=(N_TC_BLK,),
        in_specs=[
            pl.BlockSpec((1, 1, R_BLK), lambda i: (i, 0, 0)),
            pl.BlockSpec((R_BLK, D), lambda i: (i, 0)),
            pl.BlockSpec((1, D), lambda i: (0, 0)),
        ],
        out_specs=[
            pl.BlockSpec((R_BLK, D), lambda i: (i, 0)),
            pl.BlockSpec((1, B), lambda i: (0, 0)),
            pl.BlockSpec((B, D), lambda i: (0, 0)),
        ],
        out_shape=[
            jax.ShapeDtypeStruct((N, D), jnp.float32),
            jax.ShapeDtypeStruct((1, B), jnp.float32),
            jax.ShapeDtypeStruct((B, D), jnp.float32),
        ],
    )(bid3, nfeats, init_emb)

    vnfeat_out = pl.pallas_call(
        _tc_mlp_body,
        out_shape=jax.ShapeDtypeStruct((B, D), jnp.float32),
    )(pooled_sc, pooled_tc, counts, init_emb, W1, b1.reshape(1, H), W2,
      b2.reshape(1, D))

    return nfeats_out, vnfeat_out
